# baseline (device time: 88125 ns/iter reference)
import jax
import jax.numpy as jnp
from jax import lax
from jax.experimental import pallas as pl
from jax.experimental.pallas import tpu as pltpu

N_DEV = 4
SQ = 1024
SKV = 1024
HQ_LOCAL = 8
DH = 128
BLK = 64
BAND = 128
CH = 2 * BAND
LOKV = 512
HC = SKV // 2
SCALE = 0.08838834764831843


def kernel(x, Wq, K_ext, V_ext, Wo):
    my = lax.axis_index("i")
    x2 = x[0]
    K = lax.dynamic_slice_in_dim(
        K_ext[0].reshape(SKV, 32 * DH), my * HQ_LOCAL * DH,
        HQ_LOCAL * DH, axis=1)
    V = lax.dynamic_slice_in_dim(
        V_ext[0].reshape(SKV, 32 * DH), my * HQ_LOCAL * DH,
        HQ_LOCAL * DH, axis=1)

    def body(x_ref, wq_ref, k_ref, v_ref, wo_ref, out_ref,
             cw_ref, ccw_ref, cw_send, cw_recv, ccw_send, ccw_recv):
        my_pos = lax.axis_index("i")
        left = lax.rem(my_pos + N_DEV - 1, N_DEV)
        right = lax.rem(my_pos + 1, N_DEV)

        barrier_sem = pltpu.get_barrier_semaphore()
        for nbr in [left, right]:
            pl.semaphore_signal(
                barrier_sem, inc=1,
                device_id=(nbr,), device_id_type=pl.DeviceIdType.MESH,
            )
        pl.semaphore_wait(barrier_sem, 2)

        bf = jnp.bfloat16
        col_lo = lax.broadcasted_iota(jnp.int32, (BAND, LOKV), 1) // BLK
        row_lo = lax.broadcasted_iota(jnp.int32, (BAND, LOKV), 0) // BLK
        col_hi = lax.broadcasted_iota(jnp.int32, (BAND, SKV), 1) // BLK
        row_hi = lax.broadcasted_iota(jnp.int32, (BAND, SKV), 0) // BLK

        def qproj(r):
            xq = jnp.concatenate(
                [x_ref[pl.ds(r * BAND, BAND), :],
                 x_ref[pl.ds(512 + r * BAND, BAND), :]], axis=0)
            return jnp.dot(xq, wq_ref[...],
                           preferred_element_type=jnp.float32
                           ) * (SCALE * 1.4426950408889634)

        def att_out(r, qv, heads):
            mask_lo = col_lo <= row_lo + 2 * r
            mask_hi = col_hi <= row_hi + 2 * r + 8
            ctx_lo, ctx_hi = [], []
            for h in heads:
                q_lo = qv[:BAND, h * DH:(h + 1) * DH]
                q_hi = qv[BAND:, h * DH:(h + 1) * DH]
                k_h = k_ref[:, h * DH:(h + 1) * DH]
                v_h = v_ref[:, h * DH:(h + 1) * DH]
                s_lo = lax.dot_general(
                    q_lo, k_h[:LOKV], (((1,), (1,)), ((), ())),
                    preferred_element_type=jnp.float32)
                s_hi = lax.dot_general(
                    q_hi, k_h, (((1,), (1,)), ((), ())),
                    preferred_element_type=jnp.float32)
                w_lo = jnp.where(mask_lo, jnp.exp2(s_lo), 0.0)
                w_hi = jnp.where(mask_hi, jnp.exp2(s_hi), 0.0)
                d_lo = 1.0 / jnp.sum(w_lo, axis=-1, keepdims=True)
                d_hi = 1.0 / jnp.sum(w_hi, axis=-1, keepdims=True)
                c_lo = jnp.dot(w_lo, v_h[:LOKV],
                               preferred_element_type=jnp.float32) * d_lo
                c_hi = jnp.dot(w_hi, v_h,
                               preferred_element_type=jnp.float32) * d_hi
                ctx_lo.append(c_lo)
                ctx_hi.append(c_hi)
            wo_rows = wo_ref[heads[0] * DH:(heads[-1] + 1) * DH, :]
            p_lo = jnp.dot(jnp.concatenate(ctx_lo, axis=1), wo_rows,
                           preferred_element_type=jnp.float32)
            p_hi = jnp.dot(jnp.concatenate(ctx_hi, axis=1), wo_rows,
                           preferred_element_type=jnp.float32)
            return jnp.concatenate([p_lo, p_hi], axis=0)

        def pchunk(r):
            return att_out(r, qproj(r), list(range(HQ_LOCAL)))

        def store(r, src, lo, hi):
            out_ref[0, pl.ds(r * BAND, BAND), lo:hi] = src[:BAND, :]
            out_ref[0, pl.ds(512 + r * BAND, BAND), lo:hi] = src[BAND:, :]

        def hop(h, ring_ref, send_sems, recv_sems, dst):
            return pltpu.make_async_remote_copy(
                src_ref=ring_ref.at[h % 2],
                dst_ref=ring_ref.at[(h + 1) % 2],
                send_sem=send_sems.at[h],
                recv_sem=recv_sems.at[h],
                device_id=(dst,),
                device_id_type=pl.DeviceIdType.MESH,
            )

        def cw_hop(h):
            return hop(h, cw_ref, cw_send, cw_recv, right)

        def ccw_hop(h):
            return hop(h, ccw_ref, ccw_send, ccw_recv, left)

        a = lax.rem(my_pos + 3, N_DEV)
        b = lax.rem(my_pos + 1, N_DEV)
        c = lax.rem(my_pos + 2, N_DEV)

        pa = pchunk(a)
        pb = pchunk(b)
        cw_ref[0] = pa[:, :HC].astype(bf)
        ccw_ref[0] = pb[:, HC:].astype(bf)
        cw0, ccw0 = cw_hop(0), ccw_hop(0)
        cw0.start()
        ccw0.start()
        pc = pchunk(c)
        cw0.wait()
        ccw0.wait()
        cw_ref[1] = (cw_ref[1][...] + pc[:, :HC]).astype(bf)
        ccw_ref[1] = (ccw_ref[1][...] + pc[:, HC:]).astype(bf)
        cw1, ccw1 = cw_hop(1), ccw_hop(1)
        cw1.start()
        ccw1.start()
        qv_i = qproj(my_pos)
        pi1 = att_out(my_pos, qv_i, list(range(0, HQ_LOCAL // 2)))
        cw1.wait()
        ccw1.wait()
        cw_ref[0] = (cw_ref[0][...] + pb[:, :HC]).astype(bf)
        ccw_ref[0] = (ccw_ref[0][...] + pa[:, HC:]).astype(bf)
        cw2, ccw2 = cw_hop(2), ccw_hop(2)
        cw2.start()
        ccw2.start()
        pi = pi1 + att_out(my_pos, qv_i, list(range(HQ_LOCAL // 2, HQ_LOCAL)))
        cw2.wait()
        ccw2.wait()
        red_l = cw_ref[1][...] + pi[:, :HC]
        red_r = ccw_ref[1][...] + pi[:, HC:]

        cw_ref[1] = red_l.astype(bf)
        ccw_ref[1] = red_r.astype(bf)
        cw3, ccw3 = cw_hop(3), ccw_hop(3)
        cw3.start()
        ccw3.start()
        store(my_pos, red_l, 0, HC)
        store(my_pos, red_r, HC, SKV)
        cw3.wait()
        ccw3.wait()
        cw4, ccw4 = cw_hop(4), ccw_hop(4)
        cw4.start()
        ccw4.start()
        f32 = jnp.float32
        store(a, cw_ref[0][...].astype(f32), 0, HC)
        store(b, ccw_ref[0][...].astype(f32), HC, SKV)
        cw4.wait()
        ccw4.wait()
        cw5, ccw5 = cw_hop(5), ccw_hop(5)
        cw5.start()
        ccw5.start()
        store(c, cw_ref[1][...].astype(f32), 0, HC)
        store(c, ccw_ref[1][...].astype(f32), HC, SKV)
        cw5.wait()
        ccw5.wait()
        store(b, cw_ref[0][...].astype(f32), 0, HC)
        store(a, ccw_ref[0][...].astype(f32), HC, SKV)

    out = pl.pallas_call(
        body,
        out_shape=jax.ShapeDtypeStruct((1, SQ, SKV), jnp.float32),
        in_specs=[pl.BlockSpec(memory_space=pltpu.VMEM)] * 5,
        out_specs=pl.BlockSpec(memory_space=pltpu.VMEM),
        scratch_shapes=[
            pltpu.VMEM((2, CH, HC), jnp.bfloat16),
            pltpu.VMEM((2, CH, HC), jnp.bfloat16),
            pltpu.SemaphoreType.DMA((6,)),
            pltpu.SemaphoreType.DMA((6,)),
            pltpu.SemaphoreType.DMA((6,)),
            pltpu.SemaphoreType.DMA((6,)),
        ],
        compiler_params=pltpu.CompilerParams(collective_id=0),
    )(x2, Wq, K, V, Wo)
    return out


# device time: 52324 ns/iter; 1.6842x vs baseline; 1.6842x over previous
import jax
import jax.numpy as jnp
from jax import lax
from jax.experimental import pallas as pl
from jax.experimental.pallas import tpu as pltpu

N_DEV = 4
SQ = 1024
SKV = 1024
HQ_LOCAL = 8
DH = 128
BLK = 64
BAND = 128
CH = 2 * BAND
LOKV = 512
HC = SKV // 2
SCALE = 0.08838834764831843


def kernel(x, Wq, K_ext, V_ext, Wo):
    my = lax.axis_index("i")
    x2 = x[0]
    K = jnp.transpose(
        lax.dynamic_slice_in_dim(K_ext[0], my * HQ_LOCAL, HQ_LOCAL, axis=1),
        (1, 0, 2),
    )
    V = jnp.transpose(
        lax.dynamic_slice_in_dim(V_ext[0], my * HQ_LOCAL, HQ_LOCAL, axis=1),
        (1, 0, 2),
    )

    def body(x_ref, wq_ref, k_ref, v_ref, wo_ref, out_ref,
             cw_ref, ccw_ref, cw_send, cw_recv, ccw_send, ccw_recv):
        my_pos = lax.axis_index("i")
        left = lax.rem(my_pos + N_DEV - 1, N_DEV)
        right = lax.rem(my_pos + 1, N_DEV)

        barrier_sem = pltpu.get_barrier_semaphore()
        for nbr in [left, right]:
            pl.semaphore_signal(
                barrier_sem, inc=1,
                device_id=(nbr,), device_id_type=pl.DeviceIdType.MESH,
            )
        pl.semaphore_wait(barrier_sem, 2)

        bf = jnp.bfloat16
        col_lo = lax.broadcasted_iota(jnp.int32, (BAND, LOKV), 1) // BLK
        row_lo = lax.broadcasted_iota(jnp.int32, (BAND, LOKV), 0) // BLK
        col_hi = lax.broadcasted_iota(jnp.int32, (BAND, SKV), 1) // BLK
        row_hi = lax.broadcasted_iota(jnp.int32, (BAND, SKV), 0) // BLK

        def qproj(r):
            xq = jnp.concatenate(
                [x_ref[pl.ds(r * BAND, BAND), :],
                 x_ref[pl.ds(512 + r * BAND, BAND), :]], axis=0)
            return jnp.dot(xq, wq_ref[...],
                           preferred_element_type=jnp.float32
                           ) * (SCALE * 1.4426950408889634)

        def att_out(r, qv, heads):
            mask_lo = col_lo <= row_lo + 2 * r
            mask_hi = col_hi <= row_hi + 2 * r + 8
            ctx_lo, ctx_hi = [], []
            for h in heads:
                q_lo = qv[:BAND, h * DH:(h + 1) * DH]
                q_hi = qv[BAND:, h * DH:(h + 1) * DH]
                k_h = k_ref[h]
                v_h = v_ref[h]
                s_lo = lax.dot_general(
                    q_lo, k_h[:LOKV], (((1,), (1,)), ((), ())),
                    preferred_element_type=jnp.float32)
                s_hi = lax.dot_general(
                    q_hi, k_h, (((1,), (1,)), ((), ())),
                    preferred_element_type=jnp.float32)
                w_lo = jnp.where(mask_lo, jnp.exp2(s_lo), 0.0)
                w_hi = jnp.where(mask_hi, jnp.exp2(s_hi), 0.0)
                d_lo = 1.0 / jnp.sum(w_lo, axis=-1, keepdims=True)
                d_hi = 1.0 / jnp.sum(w_hi, axis=-1, keepdims=True)
                c_lo = jnp.dot(w_lo, v_h[:LOKV],
                               preferred_element_type=jnp.float32) * d_lo
                c_hi = jnp.dot(w_hi, v_h,
                               preferred_element_type=jnp.float32) * d_hi
                ctx_lo.append(c_lo)
                ctx_hi.append(c_hi)
            wo_rows = wo_ref[heads[0] * DH:(heads[-1] + 1) * DH, :]
            p_lo = jnp.dot(jnp.concatenate(ctx_lo, axis=1), wo_rows,
                           preferred_element_type=jnp.float32)
            p_hi = jnp.dot(jnp.concatenate(ctx_hi, axis=1), wo_rows,
                           preferred_element_type=jnp.float32)
            return jnp.concatenate([p_lo, p_hi], axis=0)

        def pchunk(r):
            return att_out(r, qproj(r), list(range(HQ_LOCAL)))

        def store(r, src, lo, hi):
            out_ref[0, pl.ds(r * BAND, BAND), lo:hi] = src[:BAND, :]
            out_ref[0, pl.ds(512 + r * BAND, BAND), lo:hi] = src[BAND:, :]

        def hop(h, ring_ref, send_sems, recv_sems, dst):
            return pltpu.make_async_remote_copy(
                src_ref=ring_ref.at[h % 2],
                dst_ref=ring_ref.at[(h + 1) % 2],
                send_sem=send_sems.at[h],
                recv_sem=recv_sems.at[h],
                device_id=(dst,),
                device_id_type=pl.DeviceIdType.MESH,
            )

        def cw_hop(h):
            return hop(h, cw_ref, cw_send, cw_recv, right)

        def ccw_hop(h):
            return hop(h, ccw_ref, ccw_send, ccw_recv, left)

        a = lax.rem(my_pos + 3, N_DEV)
        b = lax.rem(my_pos + 1, N_DEV)
        c = lax.rem(my_pos + 2, N_DEV)

        pa = pchunk(a)
        pb = pchunk(b)
        cw_ref[0] = pa[:, :HC].astype(bf)
        ccw_ref[0] = pb[:, HC:].astype(bf)
        cw0, ccw0 = cw_hop(0), ccw_hop(0)
        cw0.start()
        ccw0.start()
        pc = pchunk(c)
        cw0.wait()
        ccw0.wait()
        cw_ref[1] = (cw_ref[1][...] + pc[:, :HC]).astype(bf)
        ccw_ref[1] = (ccw_ref[1][...] + pc[:, HC:]).astype(bf)
        cw1, ccw1 = cw_hop(1), ccw_hop(1)
        cw1.start()
        ccw1.start()
        qv_i = qproj(my_pos)
        pi1 = att_out(my_pos, qv_i, list(range(0, HQ_LOCAL // 2)))
        cw1.wait()
        ccw1.wait()
        cw_ref[0] = (cw_ref[0][...] + pb[:, :HC]).astype(bf)
        ccw_ref[0] = (ccw_ref[0][...] + pa[:, HC:]).astype(bf)
        cw2, ccw2 = cw_hop(2), ccw_hop(2)
        cw2.start()
        ccw2.start()
        pi = pi1 + att_out(my_pos, qv_i, list(range(HQ_LOCAL // 2, HQ_LOCAL)))
        cw2.wait()
        ccw2.wait()
        red_l = cw_ref[1][...] + pi[:, :HC]
        red_r = ccw_ref[1][...] + pi[:, HC:]

        cw_ref[1] = red_l.astype(bf)
        ccw_ref[1] = red_r.astype(bf)
        cw3, ccw3 = cw_hop(3), ccw_hop(3)
        cw3.start()
        ccw3.start()
        store(my_pos, red_l, 0, HC)
        store(my_pos, red_r, HC, SKV)
        cw3.wait()
        ccw3.wait()
        cw4, ccw4 = cw_hop(4), ccw_hop(4)
        cw4.start()
        ccw4.start()
        f32 = jnp.float32
        store(a, cw_ref[0][...].astype(f32), 0, HC)
        store(b, ccw_ref[0][...].astype(f32), HC, SKV)
        cw4.wait()
        ccw4.wait()
        cw5, ccw5 = cw_hop(5), ccw_hop(5)
        cw5.start()
        ccw5.start()
        store(c, cw_ref[1][...].astype(f32), 0, HC)
        store(c, ccw_ref[1][...].astype(f32), HC, SKV)
        cw5.wait()
        ccw5.wait()
        store(b, cw_ref[0][...].astype(f32), 0, HC)
        store(a, ccw_ref[0][...].astype(f32), HC, SKV)

    out = pl.pallas_call(
        body,
        out_shape=jax.ShapeDtypeStruct((1, SQ, SKV), jnp.float32),
        in_specs=[pl.BlockSpec(memory_space=pltpu.VMEM)] * 5,
        out_specs=pl.BlockSpec(memory_space=pltpu.VMEM),
        scratch_shapes=[
            pltpu.VMEM((2, CH, HC), jnp.bfloat16),
            pltpu.VMEM((2, CH, HC), jnp.bfloat16),
            pltpu.SemaphoreType.DMA((6,)),
            pltpu.SemaphoreType.DMA((6,)),
            pltpu.SemaphoreType.DMA((6,)),
            pltpu.SemaphoreType.DMA((6,)),
        ],
        compiler_params=pltpu.CompilerParams(collective_id=0),
    )(x2, Wq, K, V, Wo)
    return out


# device time: 49927 ns/iter; 1.7651x vs baseline; 1.0480x over previous
import jax
import jax.numpy as jnp
from jax import lax
from jax.experimental import pallas as pl
from jax.experimental.pallas import tpu as pltpu

N_DEV = 4
SQ = 1024
SKV = 1024
HQ_LOCAL = 8
DH = 128
BLK = 64
BAND = 128
CH = 2 * BAND
LOKV = 512
HC = SKV // 2
SCALE = 0.08838834764831843


def kernel(x, Wq, K_ext, V_ext, Wo):
    my = lax.axis_index("i")
    x2 = x[0]
    K = jnp.transpose(
        lax.dynamic_slice_in_dim(K_ext[0], my * HQ_LOCAL, HQ_LOCAL, axis=1),
        (1, 0, 2),
    )
    V = jnp.transpose(
        lax.dynamic_slice_in_dim(V_ext[0], my * HQ_LOCAL, HQ_LOCAL, axis=1),
        (1, 0, 2),
    )

    def body(x_ref, wq_ref, k_ref, v_ref, wo_ref, out_ref,
             cw_ref, ccw_ref, cw_send, cw_recv, ccw_send, ccw_recv):
        my_pos = lax.axis_index("i")
        left = lax.rem(my_pos + N_DEV - 1, N_DEV)
        right = lax.rem(my_pos + 1, N_DEV)

        barrier_sem = pltpu.get_barrier_semaphore()
        for nbr in [left, right]:
            pl.semaphore_signal(
                barrier_sem, inc=1,
                device_id=(nbr,), device_id_type=pl.DeviceIdType.MESH,
            )
        pl.semaphore_wait(barrier_sem, 2)

        bf = jnp.bfloat16
        col_lo = lax.broadcasted_iota(jnp.int32, (BAND, LOKV), 1) // BLK
        row_lo = lax.broadcasted_iota(jnp.int32, (BAND, LOKV), 0) // BLK
        col_hi = lax.broadcasted_iota(jnp.int32, (BAND, SKV), 1) // BLK
        row_hi = lax.broadcasted_iota(jnp.int32, (BAND, SKV), 0) // BLK

        def qproj(r):
            xq = jnp.concatenate(
                [x_ref[pl.ds(r * BAND, BAND), :],
                 x_ref[pl.ds(512 + r * BAND, BAND), :]], axis=0)
            return jnp.dot(xq, wq_ref[...],
                           preferred_element_type=jnp.float32
                           ) * (SCALE * 1.4426950408889634)

        def att_out(r, qv, heads):
            mask_lo = col_lo <= row_lo + 2 * r
            mask_hi = col_hi <= row_hi + 2 * r + 8
            ctx_lo, ctx_hi = [], []
            for h in heads:
                q_lo = qv[:BAND, h * DH:(h + 1) * DH]
                q_hi = qv[BAND:, h * DH:(h + 1) * DH]
                k_h = k_ref[h]
                v_h = v_ref[h]
                s_lo = lax.dot_general(
                    q_lo, k_h[:LOKV], (((1,), (1,)), ((), ())),
                    preferred_element_type=jnp.float32)
                s_hi = lax.dot_general(
                    q_hi, k_h, (((1,), (1,)), ((), ())),
                    preferred_element_type=jnp.float32)
                w_lo = jnp.where(mask_lo, jnp.exp2(s_lo), 0.0)
                w_hi = jnp.where(mask_hi, jnp.exp2(s_hi), 0.0)
                d_lo = 1.0 / jnp.sum(w_lo, axis=-1, keepdims=True)
                d_hi = 1.0 / jnp.sum(w_hi, axis=-1, keepdims=True)
                c_lo = jnp.dot(w_lo, v_h[:LOKV],
                               preferred_element_type=jnp.float32) * d_lo
                c_hi = jnp.dot(w_hi, v_h,
                               preferred_element_type=jnp.float32) * d_hi
                ctx_lo.append(c_lo)
                ctx_hi.append(c_hi)
            wo_rows = wo_ref[heads[0] * DH:(heads[-1] + 1) * DH, :]
            p_lo = jnp.dot(jnp.concatenate(ctx_lo, axis=1), wo_rows,
                           preferred_element_type=jnp.float32)
            p_hi = jnp.dot(jnp.concatenate(ctx_hi, axis=1), wo_rows,
                           preferred_element_type=jnp.float32)
            return jnp.concatenate([p_lo, p_hi], axis=0)

        def pchunk(r):
            return att_out(r, qproj(r), list(range(HQ_LOCAL)))

        def store(r, src, lo, hi):
            out_ref[0, pl.ds(r * BAND, BAND), lo:hi] = src[:BAND, :]
            out_ref[0, pl.ds(512 + r * BAND, BAND), lo:hi] = src[BAND:, :]

        def hop(h, ring_ref, send_sems, recv_sems, dst):
            return pltpu.make_async_remote_copy(
                src_ref=ring_ref.at[h % 2],
                dst_ref=ring_ref.at[(h + 1) % 2],
                send_sem=send_sems.at[h, 0],
                recv_sem=recv_sems.at[h, 0],
                device_id=(dst,),
                device_id_type=pl.DeviceIdType.MESH,
            )

        def subhop(h, sub, ring_ref, send_sems, recv_sems, dst):
            rows = pl.ds(sub * BAND, BAND)
            return pltpu.make_async_remote_copy(
                src_ref=ring_ref.at[h % 2, rows],
                dst_ref=ring_ref.at[(h + 1) % 2, rows],
                send_sem=send_sems.at[h, sub],
                recv_sem=recv_sems.at[h, sub],
                device_id=(dst,),
                device_id_type=pl.DeviceIdType.MESH,
            )

        def cw_hop(h):
            return hop(h, cw_ref, cw_send, cw_recv, right)

        def ccw_hop(h):
            return hop(h, ccw_ref, ccw_send, ccw_recv, left)

        def cw_sub(h, sub):
            return subhop(h, sub, cw_ref, cw_send, cw_recv, right)

        def ccw_sub(h, sub):
            return subhop(h, sub, ccw_ref, ccw_send, ccw_recv, left)

        a = lax.rem(my_pos + 3, N_DEV)
        b = lax.rem(my_pos + 1, N_DEV)
        c = lax.rem(my_pos + 2, N_DEV)

        pa = pchunk(a)
        pb = pchunk(b)
        cw_ref[0] = pa[:, :HC].astype(bf)
        ccw_ref[0] = pb[:, HC:].astype(bf)
        cw0, ccw0 = cw_hop(0), ccw_hop(0)
        cw0.start()
        ccw0.start()
        pc = pchunk(c)
        cw0.wait()
        ccw0.wait()
        cw_ref[1] = (cw_ref[1][...] + pc[:, :HC]).astype(bf)
        ccw_ref[1] = (ccw_ref[1][...] + pc[:, HC:]).astype(bf)
        cw1, ccw1 = cw_hop(1), ccw_hop(1)
        cw1.start()
        ccw1.start()
        qv_i = qproj(my_pos)
        pi1 = att_out(my_pos, qv_i, list(range(0, HQ_LOCAL // 2)))
        cw1.wait()
        ccw1.wait()
        cw_ref[0] = (cw_ref[0][...] + pb[:, :HC]).astype(bf)
        ccw_ref[0] = (ccw_ref[0][...] + pa[:, HC:]).astype(bf)
        cw2, ccw2 = cw_hop(2), ccw_hop(2)
        cw2.start()
        ccw2.start()
        pi = pi1 + att_out(my_pos, qv_i, list(range(HQ_LOCAL // 2, HQ_LOCAL)))
        cw2.wait()
        ccw2.wait()
        red_l = cw_ref[1][...] + pi[:, :HC]
        red_r = ccw_ref[1][...] + pi[:, HC:]

        f32 = jnp.float32

        def store_band(r, sub, src, lo, hi):
            out_ref[0, pl.ds(512 * sub + r * BAND, BAND), lo:hi] = src

        def band_of(ring_ref, slot, sub):
            return ring_ref[slot, sub * BAND:(sub + 1) * BAND, :].astype(f32)

        cw_ref[1] = red_l.astype(bf)
        ccw_ref[1] = red_r.astype(bf)
        cwA3, cwB3 = cw_sub(3, 0), cw_sub(3, 1)
        ccwA3, ccwB3 = ccw_sub(3, 0), ccw_sub(3, 1)
        cwA3.start()
        ccwA3.start()
        cwB3.start()
        ccwB3.start()
        store(my_pos, red_l, 0, HC)
        store(my_pos, red_r, HC, SKV)
        cwA3.wait()
        cwA4 = cw_sub(4, 0)
        cwA4.start()
        ccwA3.wait()
        ccwA4 = ccw_sub(4, 0)
        ccwA4.start()
        store_band(a, 0, band_of(cw_ref, 0, 0), 0, HC)
        store_band(b, 0, band_of(ccw_ref, 0, 0), HC, SKV)
        cwB3.wait()
        cwB4 = cw_sub(4, 1)
        cwB4.start()
        ccwB3.wait()
        ccwB4 = ccw_sub(4, 1)
        ccwB4.start()
        store_band(a, 1, band_of(cw_ref, 0, 1), 0, HC)
        store_band(b, 1, band_of(ccw_ref, 0, 1), HC, SKV)
        cwA4.wait()
        cwA5 = cw_sub(5, 0)
        cwA5.start()
        ccwA4.wait()
        ccwA5 = ccw_sub(5, 0)
        ccwA5.start()
        store_band(c, 0, band_of(cw_ref, 1, 0), 0, HC)
        store_band(c, 0, band_of(ccw_ref, 1, 0), HC, SKV)
        cwB4.wait()
        cwB5 = cw_sub(5, 1)
        cwB5.start()
        ccwB4.wait()
        ccwB5 = ccw_sub(5, 1)
        ccwB5.start()
        store_band(c, 1, band_of(cw_ref, 1, 1), 0, HC)
        store_band(c, 1, band_of(ccw_ref, 1, 1), HC, SKV)
        cwA5.wait()
        ccwA5.wait()
        store_band(b, 0, band_of(cw_ref, 0, 0), 0, HC)
        store_band(a, 0, band_of(ccw_ref, 0, 0), HC, SKV)
        cwB5.wait()
        ccwB5.wait()
        store_band(b, 1, band_of(cw_ref, 0, 1), 0, HC)
        store_band(a, 1, band_of(ccw_ref, 0, 1), HC, SKV)

    out = pl.pallas_call(
        body,
        out_shape=jax.ShapeDtypeStruct((1, SQ, SKV), jnp.float32),
        in_specs=[pl.BlockSpec(memory_space=pltpu.VMEM)] * 5,
        out_specs=pl.BlockSpec(memory_space=pltpu.VMEM),
        scratch_shapes=[
            pltpu.VMEM((2, CH, HC), jnp.bfloat16),
            pltpu.VMEM((2, CH, HC), jnp.bfloat16),
            pltpu.SemaphoreType.DMA((6, 2)),
            pltpu.SemaphoreType.DMA((6, 2)),
            pltpu.SemaphoreType.DMA((6, 2)),
            pltpu.SemaphoreType.DMA((6, 2)),
        ],
        compiler_params=pltpu.CompilerParams(collective_id=0),
    )(x2, Wq, K, V, Wo)
    return out
